# baseline (speedup 1.0000x reference)
import functools

import jax
import jax.numpy as jnp
from jax import lax
from jax.experimental import pallas as pl
from jax.experimental.pallas import tpu as pltpu
from jax.experimental.pallas import tpu_sc as plsc

_EPS = 1e-5


def _gelu(t):
    return 0.5 * t * (1.0 + lax.erf(t * 0.7071067811865476))


def _bnorm(t, g, b):
    mu = jnp.mean(t, axis=0, keepdims=True)
    var = jnp.mean((t - mu) * (t - mu), axis=0, keepdims=True)
    return (t - mu) / jnp.sqrt(var + _EPS) * g + b


def _node_prep_body(gd_ref, w1_ref, b1_ref, g1_ref, be1_ref, wmsg_ref, bmsg_ref,
                    p_ref, q_ref):
    x = jnp.dot(gd_ref[...], w1_ref[...], preferred_element_type=jnp.float32)
    x = _bnorm(x + b1_ref[...], g1_ref[...], be1_ref[...])
    x = _gelu(x)
    d = x.shape[1]
    h = d // 2
    wa = wmsg_ref[:d, :]
    wb = wmsg_ref[d:, :]
    pp = jnp.dot(x, wa - wb, preferred_element_type=jnp.float32) + bmsg_ref[...]
    qq = jnp.dot(x, wb, preferred_element_type=jnp.float32)
    p_ref[0] = pp[:, :h]
    p_ref[1] = pp[:, h:]
    q_ref[0] = qq[:, :h]
    q_ref[1] = qq[:, h:]


def _gate_body(ea_ref, wg_ref, bg_ref, gate_ref):
    z = jnp.dot(ea_ref[...], wg_ref[...], preferred_element_type=jnp.float32)
    z = jax.nn.sigmoid(z + bg_ref[...])
    h = z.shape[1] // 2
    gate_ref[0] = z[:, :h]
    gate_ref[1] = z[:, h:]


def _tail_body(acc_ref, gd_ref, gbn_ref, bbn_ref, wm_ref, bm_ref, gm_ref, bem_ref,
               wl_ref, bl_ref, gl_ref, bel_ref, w2_ref, b2_ref, g2_ref, be2_ref,
               out_ref):
    npts = gd_ref.shape[0]
    agg = jnp.concatenate([acc_ref[0, :npts], acc_ref[1, :npts]], axis=1)
    y = _gelu(_bnorm(agg, gbn_ref[...], bbn_ref[...]))
    h = jnp.dot(y, wm_ref[...], preferred_element_type=jnp.float32) + bm_ref[...]
    h = _bnorm(h, gm_ref[...], bem_ref[...])
    att = jax.nn.sigmoid(jnp.max(h, axis=1, keepdims=True))
    y2 = jnp.dot(y * att, wl_ref[...], preferred_element_type=jnp.float32) + bl_ref[...]
    y2 = _bnorm(y2, gl_ref[...], bel_ref[...])
    out = jnp.dot(y2, w2_ref[...], preferred_element_type=jnp.float32) + b2_ref[...]
    out_ref[...] = _bnorm(out, g2_ref[...], be2_ref[...]) + gd_ref[...]


def _sc_edge_aggregate(p_tab, q_tab, gate, src2, dst1, n, hd):
    e = gate.shape[1]
    ncores, nsub = 2, 16
    chunk = 80
    edges_per_tile = e // nsub
    nchunks = edges_per_tile // chunk
    assert edges_per_tile * nsub == e
    assert nchunks * chunk == edges_per_tile
    assert nchunks % 2 == 0 and nchunks >= 4
    nodes_per_tile = ((n + nsub * 8 - 1) // (nsub * 8)) * 8
    n_pad = nodes_per_tile * nsub
    zeros_blk = jnp.zeros((nodes_per_tile, hd), jnp.float32)

    mesh = plsc.VectorSubcoreMesh(core_axis_name="c", subcore_axis_name="s",
                                  num_cores=ncores, num_subcores=nsub)

    @functools.partial(
        pl.kernel,
        out_type=jax.ShapeDtypeStruct((ncores, n_pad, hd), jnp.float32),
        mesh=mesh,
        compiler_params=pltpu.CompilerParams(use_tc_tiling_on_sc=False),
        scratch_types=[
            pltpu.VMEM((nchunks, chunk), jnp.int32),
            pltpu.VMEM((nchunks, chunk), jnp.int32),
            pltpu.VMEM((chunk,), jnp.int32),
            pltpu.VMEM((chunk,), jnp.int32),
            pltpu.VMEM((chunk,), jnp.int32),
            pltpu.VMEM((chunk,), jnp.int32),
            pltpu.VMEM((chunk, hd), jnp.float32),
            pltpu.VMEM((chunk, hd), jnp.float32),
            pltpu.VMEM((chunk, hd), jnp.float32),
            pltpu.VMEM((chunk, hd), jnp.float32),
            pltpu.VMEM((chunk, hd), jnp.float32),
            pltpu.VMEM((chunk, hd), jnp.float32),
            pltpu.VMEM_SHARED((n_pad, hd), jnp.float32),
            pltpu.SemaphoreType.DMA, pltpu.SemaphoreType.DMA,
            pltpu.SemaphoreType.DMA, pltpu.SemaphoreType.DMA,
            pltpu.SemaphoreType.DMA, pltpu.SemaphoreType.DMA,
        ],
    )
    def sc_kernel(p_hbm, q_hbm, gate_hbm, src_hbm, dst_hbm, z_hbm, out_hbm,
                  src_all, dst_all, dg0, dg1, dr0, dr1, p0, p1, q0, q1, g0, g1,
                  acc, sp0, sp1, sq0, sq1, sg0, sg1):
        p_v, q_v, g_v, dg_v = (p0, p1), (q0, q1), (g0, g1), (dg0, dg1)
        dr_v = (dr0, dr1)
        sem_p, sem_q, sem_g = (sp0, sp1), (sq0, sq1), (sg0, sg1)
        c = lax.axis_index("c")
        s = lax.axis_index("s")
        coff = c * n
        pltpu.sync_copy(z_hbm, acc.at[pl.ds(s * nodes_per_tile, nodes_per_tile)])
        plsc.subcore_barrier()
        pltpu.sync_copy(src_hbm.at[c, s], src_all)
        pltpu.sync_copy(dst_hbm.at[s], dst_all)
        ebase = s * edges_per_tile

        def prep_dg(i, b):
            for k in range(chunk // 16):
                sl = pl.ds(k * 16, 16)
                dv = dst_all[i, sl]
                dg_v[b][sl] = dv + coff
                dr_v[b][sl] = dv

        def issue(i, b):
            pltpu.async_copy(p_hbm.at[dg_v[b]], p_v[b], sem_p[b])
            pltpu.async_copy(q_hbm.at[src_all.at[i]], q_v[b], sem_q[b])
            pltpu.async_copy(gate_hbm.at[c, pl.ds(ebase + i * chunk, chunk)],
                             g_v[b], sem_g[b])

        def wait_gathers(b):
            pltpu.make_async_copy(p_hbm.at[dg_v[b]], p_v[b], sem_p[b]).wait()
            pltpu.make_async_copy(q_hbm.at[src_all.at[0]], q_v[b], sem_q[b]).wait()
            pltpu.make_async_copy(gate_hbm.at[c, pl.ds(0, chunk)], g_v[b],
                                  sem_g[b]).wait()

        def compute(b):
            def row_body(r, carry):
                for k in range(hd // 16):
                    sl = pl.ds(k * 16, 16)
                    g_v[b][r, sl] = (p_v[b][r, sl] + q_v[b][r, sl]) * g_v[b][r, sl]
                return carry

            lax.fori_loop(0, chunk, row_body, 0, unroll=False)

        def scatter(i, b):
            pltpu.sync_copy(g_v[b], acc.at[dr_v[b]], add=True)

        prep_dg(0, 0)
        issue(0, 0)

        def pair(j, carry):
            i0 = 2 * j
            wait_gathers(0)
            prep_dg(i0 + 1, 1)
            issue(i0 + 1, 1)
            compute(0)
            scatter(i0, 0)
            wait_gathers(1)
            prep_dg(i0 + 2, 0)
            issue(i0 + 2, 0)
            compute(1)
            scatter(i0 + 1, 1)
            return carry

        lax.fori_loop(0, nchunks // 2 - 1, pair, 0, unroll=False)

        t = nchunks - 2
        wait_gathers(0)
        prep_dg(t + 1, 1)
        issue(t + 1, 1)
        compute(0)
        scatter(t, 0)
        wait_gathers(1)
        compute(1)
        scatter(t + 1, 1)

        plsc.subcore_barrier()
        pltpu.sync_copy(acc.at[pl.ds(s * nodes_per_tile, nodes_per_tile)],
                        out_hbm.at[c, pl.ds(s * nodes_per_tile, nodes_per_tile)])

    return sc_kernel(p_tab, q_tab, gate, src2, dst1, zeros_blk)


def kernel(graph_data, edge_index, edge_attr, params):
    p = params
    n, d = graph_data.shape
    e = edge_index.shape[1]
    de = edge_attr.shape[1]
    hd = d // 2

    def row(v):
        return v.reshape(1, -1)

    p_tab, q_tab = pl.pallas_call(
        _node_prep_body,
        out_shape=[jax.ShapeDtypeStruct((2, n, hd), jnp.float32),
                   jax.ShapeDtypeStruct((2, n, hd), jnp.float32)],
    )(graph_data, p['W1'], row(p['b1']), row(p['g1']), row(p['be1']),
      p['Wmsg'], row(p['bmsg']))

    eb = 3200
    grid = e // eb
    gate = pl.pallas_call(
        _gate_body,
        grid=(grid,),
        in_specs=[pl.BlockSpec((eb, de), lambda i: (i, 0)),
                  pl.BlockSpec((de, d), lambda i: (0, 0)),
                  pl.BlockSpec((1, d), lambda i: (0, 0))],
        out_specs=pl.BlockSpec((2, eb, hd), lambda i: (0, i, 0)),
        out_shape=jax.ShapeDtypeStruct((2, e, hd), jnp.float32),
    )(edge_attr, p['Wgate'], row(p['bgate']))

    nsub = 16
    chunk = 80
    nchunks = e // (nsub * chunk)
    src = edge_index[0]
    dst = edge_index[1]
    src2 = jnp.stack([src, src + n]).reshape(2, nsub, nchunks, chunk)
    dst1 = dst.reshape(nsub, nchunks, chunk)

    acc = _sc_edge_aggregate(p_tab.reshape(2 * n, hd), q_tab.reshape(2 * n, hd),
                             gate, src2, dst1, n, hd)

    out = pl.pallas_call(
        _tail_body,
        out_shape=jax.ShapeDtypeStruct((n, d), jnp.float32),
    )(acc, graph_data, row(p['gbn']), row(p['bbn']), p['Wm'], row(p['bm']),
      row(p['gm']), row(p['bem']), p['Wl'], row(p['bl']), row(p['gl']),
      row(p['bel']), p['W2'], row(p['b2']), row(p['g2']), row(p['be2']))
    return out


# speedup vs baseline: 1.0974x; 1.0974x over previous
import functools

import jax
import jax.numpy as jnp
from jax import lax
from jax.experimental import pallas as pl
from jax.experimental.pallas import tpu as pltpu
from jax.experimental.pallas import tpu_sc as plsc

_EPS = 1e-5


def _gelu(t):
    return 0.5 * t * (1.0 + lax.erf(t * 0.7071067811865476))


def _bnorm(t, g, b):
    mu = jnp.mean(t, axis=0, keepdims=True)
    var = jnp.mean((t - mu) * (t - mu), axis=0, keepdims=True)
    return (t - mu) / jnp.sqrt(var + _EPS) * g + b


def _node_prep_body(gd_ref, w1_ref, b1_ref, g1_ref, be1_ref, wmsg_ref, bmsg_ref,
                    p_ref, q_ref):
    x = jnp.dot(gd_ref[...], w1_ref[...], preferred_element_type=jnp.float32)
    x = _bnorm(x + b1_ref[...], g1_ref[...], be1_ref[...])
    x = _gelu(x)
    d = x.shape[1]
    wa = wmsg_ref[:d, :]
    wb = wmsg_ref[d:, :]
    p_ref[...] = jnp.dot(x, wa - wb, preferred_element_type=jnp.float32) + bmsg_ref[...]
    q_ref[...] = jnp.dot(x, wb, preferred_element_type=jnp.float32)


def _gate_body(ea_ref, wg_ref, bg_ref, gate_ref):
    z = jnp.dot(ea_ref[...], wg_ref[...], preferred_element_type=jnp.float32)
    gate_ref[...] = jax.nn.sigmoid(z + bg_ref[...])


def _tail_body(acc_ref, gd_ref, gbn_ref, bbn_ref, wm_ref, bm_ref, gm_ref, bem_ref,
               wl_ref, bl_ref, gl_ref, bel_ref, w2_ref, b2_ref, g2_ref, be2_ref,
               out_ref):
    npts = gd_ref.shape[0]
    agg = acc_ref[0, :npts] + acc_ref[1, :npts]
    y = _gelu(_bnorm(agg, gbn_ref[...], bbn_ref[...]))
    h = jnp.dot(y, wm_ref[...], preferred_element_type=jnp.float32) + bm_ref[...]
    h = _bnorm(h, gm_ref[...], bem_ref[...])
    att = jax.nn.sigmoid(jnp.max(h, axis=1, keepdims=True))
    y2 = jnp.dot(y * att, wl_ref[...], preferred_element_type=jnp.float32) + bl_ref[...]
    y2 = _bnorm(y2, gl_ref[...], bel_ref[...])
    out = jnp.dot(y2, w2_ref[...], preferred_element_type=jnp.float32) + b2_ref[...]
    out_ref[...] = _bnorm(out, g2_ref[...], be2_ref[...]) + gd_ref[...]


def _sc_edge_aggregate(p_nodes, q_nodes, gate, src, dst):
    n, d = p_nodes.shape
    e = src.shape[0]
    ncores, nsub = 2, 16
    nw = ncores * nsub
    chunk = 16
    edges_per_tile = e // nw
    nchunks = edges_per_tile // chunk
    assert nchunks * chunk * nw == e
    assert nchunks % 2 == 1 and nchunks >= 5
    nodes_per_tile = ((n + nsub * 8 - 1) // (nsub * 8)) * 8
    n_pad = nodes_per_tile * nsub
    zeros_blk = jnp.zeros((nodes_per_tile, d), jnp.float32)

    mesh = plsc.VectorSubcoreMesh(core_axis_name="c", subcore_axis_name="s",
                                  num_cores=ncores, num_subcores=nsub)

    @functools.partial(
        pl.kernel,
        out_type=jax.ShapeDtypeStruct((ncores, n_pad, d), jnp.float32),
        mesh=mesh,
        scratch_types=[
            pltpu.VMEM((chunk,), jnp.int32),
            pltpu.VMEM((chunk,), jnp.int32),
            pltpu.VMEM((chunk,), jnp.int32),
            pltpu.VMEM((chunk,), jnp.int32),
            pltpu.VMEM((chunk,), jnp.int32),
            pltpu.VMEM((chunk,), jnp.int32),
            pltpu.VMEM((chunk, d), jnp.float32),
            pltpu.VMEM((chunk, d), jnp.float32),
            pltpu.VMEM((chunk, d), jnp.float32),
            pltpu.VMEM((chunk, d), jnp.float32),
            pltpu.VMEM((chunk, d), jnp.float32),
            pltpu.VMEM((chunk, d), jnp.float32),
            pltpu.VMEM((chunk, d), jnp.float32),
            pltpu.VMEM((chunk, d), jnp.float32),
            pltpu.VMEM_SHARED((n_pad, d), jnp.float32),
            pltpu.SemaphoreType.DMA, pltpu.SemaphoreType.DMA,
            pltpu.SemaphoreType.DMA, pltpu.SemaphoreType.DMA,
            pltpu.SemaphoreType.DMA, pltpu.SemaphoreType.DMA,
            pltpu.SemaphoreType.DMA, pltpu.SemaphoreType.DMA,
            pltpu.SemaphoreType.DMA, pltpu.SemaphoreType.DMA,
        ],
    )
    def sc_kernel(p_hbm, q_hbm, gate_hbm, src_hbm, dst_hbm, z_hbm, out_hbm,
                  sb0, sb1, db0, db1, dr0, dr1, p0, p1, q0, q1, g0, g1,
                  pr0, pr1, acc, sp0, sp1, sq0, sq1, sg0, sg1, si0, si1,
                  ss0, ss1):
        src_b, dst_b, dr_v = (sb0, sb1), (db0, db1), (dr0, dr1)
        p_v, q_v, g_v, pr_v = (p0, p1), (q0, q1), (g0, g1), (pr0, pr1)
        sem_p, sem_q, sem_g = (sp0, sp1), (sq0, sq1), (sg0, sg1)
        sem_i, sem_s = (si0, si1), (ss0, ss1)
        c = lax.axis_index("c")
        s = lax.axis_index("s")
        w = c * nsub + s
        pltpu.sync_copy(z_hbm, acc.at[pl.ds(s * nodes_per_tile, nodes_per_tile)])
        plsc.subcore_barrier()
        ebase = w * edges_per_tile

        def issue_idx(i, b):
            e0 = ebase + i * chunk
            pltpu.async_copy(src_hbm.at[pl.ds(e0, chunk)], src_b[b], sem_i[b])
            pltpu.async_copy(dst_hbm.at[pl.ds(e0, chunk)], dst_b[b], sem_i[b])

        def wait_idx(b):
            pltpu.make_async_copy(src_hbm.at[pl.ds(0, chunk)], src_b[b],
                                  sem_i[b]).wait()
            pltpu.make_async_copy(dst_hbm.at[pl.ds(0, chunk)], dst_b[b],
                                  sem_i[b]).wait()

        def issue(i, b):
            pltpu.async_copy(p_hbm.at[dst_b[b]], p_v[b], sem_p[b])
            pltpu.async_copy(q_hbm.at[src_b[b]], q_v[b], sem_q[b])
            pltpu.async_copy(gate_hbm.at[pl.ds(ebase + i * chunk, chunk)],
                             g_v[b], sem_g[b])

        def wait_gathers(b):
            pltpu.make_async_copy(p_hbm.at[dst_b[b]], p_v[b], sem_p[b]).wait()
            pltpu.make_async_copy(q_hbm.at[src_b[b]], q_v[b], sem_q[b]).wait()
            pltpu.make_async_copy(gate_hbm.at[pl.ds(0, chunk)], g_v[b],
                                  sem_g[b]).wait()

        def compute(b):
            def row_body(r, carry):
                for k in range(d // 16):
                    sl = pl.ds(k * 16, 16)
                    pr_v[b][r, sl] = (p_v[b][r, sl] + q_v[b][r, sl]) * g_v[b][r, sl]
                return carry

            lax.fori_loop(0, chunk, row_body, 0, unroll=False)

        def scatter(b):
            pltpu.async_copy(pr_v[b], acc.at[dr_v[b]], sem_s[b], add=True)

        def wait_scatter(b):
            pltpu.make_async_copy(pr_v[b], acc.at[dr_v[b]], sem_s[b]).wait()

        def step(i, b, first, last2, last1):
            wait_gathers(b)
            if not first:
                wait_scatter(b)
            dr_v[b][...] = dst_b[b][...]
            if not last1:
                wait_idx(b ^ 1)
                issue(i + 1, b ^ 1)
            if not last2:
                issue_idx(i + 2, b)
            compute(b)
            scatter(b)

        pltpu.sync_copy(src_hbm.at[pl.ds(ebase, chunk)], src_b[0])
        pltpu.sync_copy(dst_hbm.at[pl.ds(ebase, chunk)], dst_b[0])
        issue(0, 0)
        issue_idx(1, 1)
        step(0, 0, True, False, False)
        step(1, 1, True, False, False)

        def pair(j, carry):
            i0 = 2 * j
            step(i0, 0, False, False, False)
            step(i0 + 1, 1, False, False, False)
            return carry

        lax.fori_loop(1, (nchunks - 3) // 2, pair, 0, unroll=False)

        t = nchunks - 3
        step(t, 0, False, False, False)
        step(t + 1, 1, False, True, False)
        step(t + 2, 0, False, True, True)
        wait_scatter(1)
        wait_scatter(0)

        plsc.subcore_barrier()
        pltpu.sync_copy(acc.at[pl.ds(s * nodes_per_tile, nodes_per_tile)],
                        out_hbm.at[c, pl.ds(s * nodes_per_tile, nodes_per_tile)])

    return sc_kernel(p_nodes, q_nodes, gate, src, dst, zeros_blk)


def kernel(graph_data, edge_index, edge_attr, params):
    p = params
    n, d = graph_data.shape
    e = edge_index.shape[1]
    de = edge_attr.shape[1]

    def row(v):
        return v.reshape(1, -1)

    p_nodes, q_nodes = pl.pallas_call(
        _node_prep_body,
        out_shape=[jax.ShapeDtypeStruct((n, d), jnp.float32),
                   jax.ShapeDtypeStruct((n, d), jnp.float32)],
    )(graph_data, p['W1'], row(p['b1']), row(p['g1']), row(p['be1']),
      p['Wmsg'], row(p['bmsg']))

    eb = 3200
    grid = e // eb
    gate = pl.pallas_call(
        _gate_body,
        grid=(grid,),
        in_specs=[pl.BlockSpec((eb, de), lambda i: (i, 0)),
                  pl.BlockSpec((de, d), lambda i: (0, 0)),
                  pl.BlockSpec((1, d), lambda i: (0, 0))],
        out_specs=pl.BlockSpec((eb, d), lambda i: (i, 0)),
        out_shape=jax.ShapeDtypeStruct((e, d), jnp.float32),
    )(edge_attr, p['Wgate'], row(p['bgate']))

    acc = _sc_edge_aggregate(p_nodes, q_nodes, gate, edge_index[0], edge_index[1])

    out = pl.pallas_call(
        _tail_body,
        out_shape=jax.ShapeDtypeStruct((n, d), jnp.float32),
    )(acc, graph_data, row(p['gbn']), row(p['bbn']), p['Wm'], row(p['bm']),
      row(p['gm']), row(p['bem']), p['Wl'], row(p['bl']), row(p['gl']),
      row(p['bel']), p['W2'], row(p['b2']), row(p['g2']), row(p['be2']))
    return out


# speedup vs baseline: 1.1515x; 1.0493x over previous
import functools

import jax
import jax.numpy as jnp
from jax import lax
from jax.experimental import pallas as pl
from jax.experimental.pallas import tpu as pltpu
from jax.experimental.pallas import tpu_sc as plsc

_EPS = 1e-5


def _gelu(t):
    return 0.5 * t * (1.0 + lax.erf(t * 0.7071067811865476))


def _bnorm(t, g, b):
    mu = jnp.mean(t, axis=0, keepdims=True)
    var = jnp.mean((t - mu) * (t - mu), axis=0, keepdims=True)
    return (t - mu) / jnp.sqrt(var + _EPS) * g + b


def _node_prep_body(gd_ref, w1_ref, b1_ref, g1_ref, be1_ref, wmsg_ref, bmsg_ref,
                    p_ref, q_ref):
    x = jnp.dot(gd_ref[...], w1_ref[...], preferred_element_type=jnp.float32)
    x = _bnorm(x + b1_ref[...], g1_ref[...], be1_ref[...])
    x = _gelu(x)
    d = x.shape[1]
    wa = wmsg_ref[:d, :]
    wb = wmsg_ref[d:, :]
    p_ref[...] = jnp.dot(x, wa - wb, preferred_element_type=jnp.float32) + bmsg_ref[...]
    q_ref[...] = jnp.dot(x, wb, preferred_element_type=jnp.float32)


def _gate_body(ea_ref, wg_ref, bg_ref, gate_ref):
    z = jnp.dot(ea_ref[...], wg_ref[...], preferred_element_type=jnp.float32)
    gate_ref[...] = jax.nn.sigmoid(z + bg_ref[...])


def _tail_body(acc_ref, gd_ref, gbn_ref, bbn_ref, wm_ref, bm_ref, gm_ref, bem_ref,
               wl_ref, bl_ref, gl_ref, bel_ref, w2_ref, b2_ref, g2_ref, be2_ref,
               out_ref):
    npts = gd_ref.shape[0]
    agg = acc_ref[0, :npts] + acc_ref[1, :npts]
    y = _gelu(_bnorm(agg, gbn_ref[...], bbn_ref[...]))
    h = jnp.dot(y, wm_ref[...], preferred_element_type=jnp.float32) + bm_ref[...]
    h = _bnorm(h, gm_ref[...], bem_ref[...])
    att = jax.nn.sigmoid(jnp.max(h, axis=1, keepdims=True))
    y2 = jnp.dot(y * att, wl_ref[...], preferred_element_type=jnp.float32) + bl_ref[...]
    y2 = _bnorm(y2, gl_ref[...], bel_ref[...])
    out = jnp.dot(y2, w2_ref[...], preferred_element_type=jnp.float32) + b2_ref[...]
    out_ref[...] = _bnorm(out, g2_ref[...], be2_ref[...]) + gd_ref[...]


def _sc_edge_aggregate(p_nodes, q_nodes, gate, src, dst):
    n, d = p_nodes.shape
    e = src.shape[0]
    ncores, nsub = 2, 16
    nw = ncores * nsub
    chunk = 16
    edges_per_tile = e // nw
    nchunks = edges_per_tile // chunk
    assert nchunks * chunk * nw == e
    assert nchunks % 2 == 1 and nchunks >= 5
    nodes_per_tile = ((n + nsub * 8 - 1) // (nsub * 8)) * 8
    n_pad = nodes_per_tile * nsub
    zeros_blk = jnp.zeros((nodes_per_tile, d), jnp.float32)

    mesh = plsc.VectorSubcoreMesh(core_axis_name="c", subcore_axis_name="s",
                                  num_cores=ncores, num_subcores=nsub)

    @functools.partial(
        pl.kernel,
        out_type=jax.ShapeDtypeStruct((ncores, n_pad, d), jnp.float32),
        mesh=mesh,
        scratch_types=(
            [pltpu.VMEM((chunk,), jnp.int32)] * 8 +
            [pltpu.VMEM((chunk, d), jnp.float32)] * 12 +
            [pltpu.VMEM_SHARED((n_pad, d), jnp.float32)] +
            [pltpu.SemaphoreType.DMA] * 16
        ),
    )
    def sc_kernel(p_hbm, q_hbm, gate_hbm, src_hbm, dst_hbm, z_hbm, out_hbm,
                  *refs):
        src_b, dst_b = refs[0:4], refs[4:8]
        p_v, q_v, g_v = refs[8:12], refs[12:16], refs[16:20]
        acc = refs[20]
        sem_p, sem_q = refs[21:25], refs[25:29]
        sem_g, sem_i = refs[29:33], refs[33:37]
        c = lax.axis_index("c")
        s = lax.axis_index("s")
        w = c * nsub + s
        pltpu.sync_copy(z_hbm, acc.at[pl.ds(s * nodes_per_tile, nodes_per_tile)])
        plsc.subcore_barrier()
        ebase = w * edges_per_tile

        def issue_idx(i, b):
            e0 = ebase + i * chunk
            pltpu.async_copy(src_hbm.at[pl.ds(e0, chunk)], src_b[b], sem_i[b])
            pltpu.async_copy(dst_hbm.at[pl.ds(e0, chunk)], dst_b[b], sem_i[b])

        def wait_idx(b):
            pltpu.make_async_copy(src_hbm.at[pl.ds(0, chunk)], src_b[b],
                                  sem_i[b]).wait()
            pltpu.make_async_copy(dst_hbm.at[pl.ds(0, chunk)], dst_b[b],
                                  sem_i[b]).wait()

        def issue(i, b):
            pltpu.async_copy(p_hbm.at[dst_b[b]], p_v[b], sem_p[b])
            pltpu.async_copy(q_hbm.at[src_b[b]], q_v[b], sem_q[b])
            pltpu.async_copy(gate_hbm.at[pl.ds(ebase + i * chunk, chunk)],
                             g_v[b], sem_g[b])

        def wait_gathers(b):
            pltpu.make_async_copy(p_hbm.at[dst_b[b]], p_v[b], sem_p[b]).wait()
            pltpu.make_async_copy(q_hbm.at[src_b[b]], q_v[b], sem_q[b]).wait()
            pltpu.make_async_copy(gate_hbm.at[pl.ds(0, chunk)], g_v[b],
                                  sem_g[b]).wait()

        def compute(b):
            def row_body(r, carry):
                for k in range(d // 16):
                    sl = pl.ds(k * 16, 16)
                    g_v[b][r, sl] = (p_v[b][r, sl] + q_v[b][r, sl]) * g_v[b][r, sl]
                return carry

            lax.fori_loop(0, chunk, row_body, 0, unroll=False)

        def scatter(b):
            pltpu.sync_copy(g_v[b], acc.at[dst_b[b]], add=True)

        def step(i, b, pre_gather, pre_idx):
            wait_gathers(b)
            if pre_gather:
                wait_idx((b + 3) % 4)
                issue(i + 3, (b + 3) % 4)
            compute(b)
            scatter(b)
            if pre_idx:
                issue_idx(i + 4, b)

        for b in range(4):
            issue_idx(b, b)
        for b in range(3):
            wait_idx(b)
            issue(b, b)

        def quad(j, carry):
            i0 = 4 * j
            for b in range(4):
                step(i0 + b, b, True, True)
            return carry

        n_quads = (nchunks - 4) // 4
        lax.fori_loop(0, n_quads, quad, 0, unroll=False)

        for i in range(4 * n_quads, nchunks):
            step(i, i % 4, i + 3 < nchunks, i + 4 < nchunks)

        plsc.subcore_barrier()
        pltpu.sync_copy(acc.at[pl.ds(s * nodes_per_tile, nodes_per_tile)],
                        out_hbm.at[c, pl.ds(s * nodes_per_tile, nodes_per_tile)])

    return sc_kernel(p_nodes, q_nodes, gate, src, dst, zeros_blk)


def kernel(graph_data, edge_index, edge_attr, params):
    p = params
    n, d = graph_data.shape
    e = edge_index.shape[1]
    de = edge_attr.shape[1]

    def row(v):
        return v.reshape(1, -1)

    p_nodes, q_nodes = pl.pallas_call(
        _node_prep_body,
        out_shape=[jax.ShapeDtypeStruct((n, d), jnp.float32),
                   jax.ShapeDtypeStruct((n, d), jnp.float32)],
    )(graph_data, p['W1'], row(p['b1']), row(p['g1']), row(p['be1']),
      p['Wmsg'], row(p['bmsg']))

    eb = 3200
    grid = e // eb
    gate = pl.pallas_call(
        _gate_body,
        grid=(grid,),
        in_specs=[pl.BlockSpec((eb, de), lambda i: (i, 0)),
                  pl.BlockSpec((de, d), lambda i: (0, 0)),
                  pl.BlockSpec((1, d), lambda i: (0, 0))],
        out_specs=pl.BlockSpec((eb, d), lambda i: (i, 0)),
        out_shape=jax.ShapeDtypeStruct((e, d), jnp.float32),
    )(edge_attr, p['Wgate'], row(p['bgate']))

    acc = _sc_edge_aggregate(p_nodes, q_nodes, gate, edge_index[0], edge_index[1])

    out = pl.pallas_call(
        _tail_body,
        out_shape=jax.ShapeDtypeStruct((n, d), jnp.float32),
    )(acc, graph_data, row(p['gbn']), row(p['bbn']), p['Wm'], row(p['bm']),
      row(p['gm']), row(p['bem']), p['Wl'], row(p['bl']), row(p['gl']),
      row(p['bel']), p['W2'], row(p['b2']), row(p['g2']), row(p['be2']))
    return out
